# baseline (device time: 62283 ns/iter reference)
import jax
import jax.numpy as jnp
from jax import lax
from jax.experimental import pallas as pl
from jax.experimental.pallas import tpu as pltpu

N_DEV = 32
M_PER = 32


def kernel(x, w_mat):
    k_loc, n = w_mat.shape
    m = x.shape[0]

    def body(x_ref, w_ref, out_ref, sendbuf_ref, inbox_ref, send_sems, recv_sems):
        my = lax.axis_index("i")

        bar = pltpu.get_barrier_semaphore()
        for off in range(1, N_DEV):
            pl.semaphore_signal(
                bar, inc=1,
                device_id=(lax.rem(my + off, N_DEV),),
                device_id_type=pl.DeviceIdType.MESH,
            )
        pl.semaphore_wait(bar, N_DEV - 1)

        rdmas = []
        for j in range(1, N_DEV):
            tgt = lax.rem(my + j, N_DEV)
            sendbuf_ref[j - 1] = jnp.dot(
                x_ref[pl.ds(tgt * M_PER, M_PER), :],
                w_ref[...],
                preferred_element_type=jnp.float32,
            )
            rdma = pltpu.make_async_remote_copy(
                src_ref=sendbuf_ref.at[j - 1],
                dst_ref=inbox_ref.at[j - 1],
                send_sem=send_sems.at[j - 1],
                recv_sem=recv_sems.at[j - 1],
                device_id=(tgt,),
                device_id_type=pl.DeviceIdType.MESH,
            )
            rdma.start()
            rdmas.append(rdma)

        result = jnp.dot(
            x_ref[pl.ds(my * M_PER, M_PER), :],
            w_ref[...],
            preferred_element_type=jnp.float32,
        )
        for j in range(1, N_DEV):
            rdmas[j - 1].wait_recv()
            result = result + inbox_ref[j - 1]
        out_ref[...] = jnp.maximum(result, 0.0)

        for j in range(1, N_DEV):
            rdmas[j - 1].wait_send()

    return pl.pallas_call(
        body,
        out_shape=jax.ShapeDtypeStruct((M_PER, n), jnp.float32),
        in_specs=[
            pl.BlockSpec(memory_space=pltpu.VMEM),
            pl.BlockSpec(memory_space=pltpu.VMEM),
        ],
        out_specs=pl.BlockSpec(memory_space=pltpu.VMEM),
        scratch_shapes=[
            pltpu.VMEM((N_DEV - 1, M_PER, n), jnp.float32),
            pltpu.VMEM((N_DEV - 1, M_PER, n), jnp.float32),
            pltpu.SemaphoreType.DMA((N_DEV - 1,)),
            pltpu.SemaphoreType.DMA((N_DEV - 1,)),
        ],
        compiler_params=pltpu.CompilerParams(collective_id=0),
    )(x, w_mat)


# device time: 43711 ns/iter; 1.4249x vs baseline; 1.4249x over previous
import jax
import jax.numpy as jnp
from jax import lax
from jax.experimental import pallas as pl
from jax.experimental.pallas import tpu as pltpu

N_DEV = 32
M_PER = 32
SQ = 4
NG = 8


def kernel(x, w_mat):
    k_loc, n = w_mat.shape
    m = x.shape[0]

    def body(x_ref, w_ref, out_ref, sbuf1_ref, work_ref, inbox1_ref,
             inbox2_ref, send1, recv1, send2, recv2):
        my = lax.axis_index("i")
        q = lax.rem(my, SQ)
        s = my // SQ
        base = my - q

        bar = pltpu.get_barrier_semaphore()
        for off in range(1, N_DEV):
            pl.semaphore_signal(
                bar, inc=1,
                device_id=(lax.rem(my + off, N_DEV),),
                device_id_type=pl.DeviceIdType.MESH,
            )
        pl.semaphore_wait(bar, N_DEV - 1)

        rdma1 = []
        for v in range(1, SQ):
            qv = lax.rem(q + v, SQ)
            for t in range(NG):
                sbuf1_ref[v - 1, pl.ds(t * M_PER, M_PER), :] = jnp.dot(
                    x_ref[pl.ds((qv + t * SQ) * M_PER, M_PER), :],
                    w_ref[...],
                    preferred_element_type=jnp.float32,
                )
            rdma = pltpu.make_async_remote_copy(
                src_ref=sbuf1_ref.at[v - 1],
                dst_ref=inbox1_ref.at[SQ - 1 - v],
                send_sem=send1.at[v - 1],
                recv_sem=recv1.at[SQ - 1 - v],
                device_id=(base + qv,),
                device_id_type=pl.DeviceIdType.MESH,
            )
            rdma.start()
            rdma1.append(rdma)

        for t in range(NG):
            work_ref[pl.ds(t * M_PER, M_PER), :] = jnp.dot(
                x_ref[pl.ds((q + t * SQ) * M_PER, M_PER), :],
                w_ref[...],
                preferred_element_type=jnp.float32,
            )

        acc = work_ref[...]
        for v in range(1, SQ):
            rdma1[v - 1].wait_recv()
            acc = acc + inbox1_ref[SQ - 1 - v]
        work_ref[...] = acc

        rdma2 = []
        for u in range(1, NG):
            t = lax.rem(s + u, NG)
            rdma = pltpu.make_async_remote_copy(
                src_ref=work_ref.at[pl.ds(t * M_PER, M_PER), :],
                dst_ref=inbox2_ref.at[NG - 1 - u],
                send_sem=send2.at[u - 1],
                recv_sem=recv2.at[NG - 1 - u],
                device_id=(t * SQ + q,),
                device_id_type=pl.DeviceIdType.MESH,
            )
            rdma.start()
            rdma2.append(rdma)

        result = work_ref[pl.ds(s * M_PER, M_PER), :]
        for u in range(1, NG):
            rdma2[u - 1].wait_recv()
            result = result + inbox2_ref[NG - 1 - u]
        out_ref[...] = jnp.maximum(result, 0.0)

        for r in rdma1 + rdma2:
            r.wait_send()

    return pl.pallas_call(
        body,
        out_shape=jax.ShapeDtypeStruct((M_PER, n), jnp.float32),
        in_specs=[
            pl.BlockSpec(memory_space=pltpu.VMEM),
            pl.BlockSpec(memory_space=pltpu.VMEM),
        ],
        out_specs=pl.BlockSpec(memory_space=pltpu.VMEM),
        scratch_shapes=[
            pltpu.VMEM((SQ - 1, NG * M_PER, n), jnp.float32),
            pltpu.VMEM((NG * M_PER, n), jnp.float32),
            pltpu.VMEM((SQ - 1, NG * M_PER, n), jnp.float32),
            pltpu.VMEM((NG - 1, M_PER, n), jnp.float32),
            pltpu.SemaphoreType.DMA((SQ - 1,)),
            pltpu.SemaphoreType.DMA((SQ - 1,)),
            pltpu.SemaphoreType.DMA((NG - 1,)),
            pltpu.SemaphoreType.DMA((NG - 1,)),
        ],
        compiler_params=pltpu.CompilerParams(collective_id=0),
    )(x, w_mat)


# device time: 37959 ns/iter; 1.6408x vs baseline; 1.1515x over previous
import jax
import jax.numpy as jnp
from jax import lax
from jax.experimental import pallas as pl
from jax.experimental.pallas import tpu as pltpu

N_DEV = 32
M_PER = 32
SQ = 4
NG = 8
H = NG // 2


def kernel(x, w_mat):
    k_loc, n = w_mat.shape

    def body(x_ref, w_ref, out_ref, sbuf1_ref, work_ref, inbox1_ref,
             inbox2_ref, send1, recv1, send2, recv2):
        my = lax.axis_index("i")
        q = lax.rem(my, SQ)
        s = my // SQ
        base = my - q

        bar = pltpu.get_barrier_semaphore()
        for off in range(1, N_DEV):
            pl.semaphore_signal(
                bar, inc=1,
                device_id=(lax.rem(my + off, N_DEV),),
                device_id_type=pl.DeviceIdType.MESH,
            )

        def slot(p):
            return lax.rem(s + 1 + p, NG)

        rdma1 = {}
        for h in range(2):
            for v in range(1, SQ):
                qv = lax.rem(q + v, SQ)
                for p in range(h * H, (h + 1) * H):
                    t = slot(p)
                    sbuf1_ref[v - 1, h,
                              pl.ds((p - h * H) * M_PER, M_PER), :] = jnp.dot(
                        x_ref[pl.ds((t * SQ + qv) * M_PER, M_PER), :],
                        w_ref[...],
                        preferred_element_type=jnp.float32,
                    )
                if h == 0 and v == 1:
                    pl.semaphore_wait(bar, N_DEV - 1)
                rdma = pltpu.make_async_remote_copy(
                    src_ref=sbuf1_ref.at[v - 1, h],
                    dst_ref=inbox1_ref.at[SQ - 1 - v, h],
                    send_sem=send1.at[v - 1, h],
                    recv_sem=recv1.at[SQ - 1 - v, h],
                    device_id=(base + qv,),
                    device_id_type=pl.DeviceIdType.MESH,
                )
                rdma.start()
                rdma1[(v, h)] = rdma

        for p in range(NG):
            work_ref[pl.ds(p * M_PER, M_PER), :] = jnp.dot(
                x_ref[pl.ds((slot(p) * SQ + q) * M_PER, M_PER), :],
                w_ref[...],
                preferred_element_type=jnp.float32,
            )

        rdma2 = {}
        for h in range(2):
            acc = work_ref[pl.ds(h * H * M_PER, H * M_PER), :]
            for v in range(1, SQ):
                rdma1[(v, h)].wait_recv()
                acc = acc + inbox1_ref[SQ - 1 - v, h]
            work_ref[pl.ds(h * H * M_PER, H * M_PER), :] = acc
            for u in range(1 + h * H, min(1 + (h + 1) * H, NG)):
                rdma = pltpu.make_async_remote_copy(
                    src_ref=work_ref.at[pl.ds((u - 1) * M_PER, M_PER), :],
                    dst_ref=inbox2_ref.at[NG - 1 - u],
                    send_sem=send2.at[u - 1],
                    recv_sem=recv2.at[NG - 1 - u],
                    device_id=(slot(u - 1) * SQ + q,),
                    device_id_type=pl.DeviceIdType.MESH,
                )
                rdma.start()
                rdma2[u] = rdma

        result = work_ref[pl.ds((NG - 1) * M_PER, M_PER), :]
        for u in range(1, NG):
            rdma2[u].wait_recv()
            result = result + inbox2_ref[NG - 1 - u]
        out_ref[...] = jnp.maximum(result, 0.0)

        for r in list(rdma1.values()) + list(rdma2.values()):
            r.wait_send()

    return pl.pallas_call(
        body,
        out_shape=jax.ShapeDtypeStruct((M_PER, n), jnp.float32),
        in_specs=[
            pl.BlockSpec(memory_space=pltpu.VMEM),
            pl.BlockSpec(memory_space=pltpu.VMEM),
        ],
        out_specs=pl.BlockSpec(memory_space=pltpu.VMEM),
        scratch_shapes=[
            pltpu.VMEM((SQ - 1, 2, H * M_PER, n), jnp.float32),
            pltpu.VMEM((NG * M_PER, n), jnp.float32),
            pltpu.VMEM((SQ - 1, 2, H * M_PER, n), jnp.float32),
            pltpu.VMEM((NG - 1, M_PER, n), jnp.float32),
            pltpu.SemaphoreType.DMA((SQ - 1, 2)),
            pltpu.SemaphoreType.DMA((SQ - 1, 2)),
            pltpu.SemaphoreType.DMA((NG - 1,)),
            pltpu.SemaphoreType.DMA((NG - 1,)),
        ],
        compiler_params=pltpu.CompilerParams(collective_id=0),
    )(x, w_mat)


# device time: 34519 ns/iter; 1.8043x vs baseline; 1.0997x over previous
import jax
import jax.numpy as jnp
from jax import lax
from jax.experimental import pallas as pl
from jax.experimental.pallas import tpu as pltpu

N_DEV = 32
M_PER = 32
SQ = 4
NG = 8
H = NG // 2


def kernel(x, w_mat):
    k_loc, n = w_mat.shape

    def body(x_ref, w_ref, out_ref, drx_ref, dry_ref, mb_ref, mc_ref,
             work_ref, in_rx_ref, in_ry_ref, in_ma1_ref, in_ma2_ref,
             inbox2_ref, s_rx, r_rx, s_ry, r_ry, s_mb, r_ma1, s_mc,
             r_ma2, send2, recv2):
        my = lax.axis_index("i")
        q = lax.rem(my, SQ)
        s = my // SQ
        base = my - q
        q1 = q ^ 1
        q2 = 3 - q
        qd = q ^ 2
        dev1 = base + q1
        dev2 = base + q2

        bar = pltpu.get_barrier_semaphore()
        for off in range(1, N_DEV):
            pl.semaphore_signal(
                bar, inc=1,
                device_id=(lax.rem(my + off, N_DEV),),
                device_id_type=pl.DeviceIdType.MESH,
            )

        def slot(p):
            return lax.rem(s + 1 + p, NG)

        def cdot(c):
            return jnp.dot(
                x_ref[pl.ds(c * M_PER, M_PER), :], w_ref[...],
                preferred_element_type=jnp.float32,
            )

        def copy(src, dst, ssem, rsem, dev):
            r = pltpu.make_async_remote_copy(
                src_ref=src, dst_ref=dst, send_sem=ssem, recv_sem=rsem,
                device_id=(dev,), device_id_type=pl.DeviceIdType.MESH,
            )
            r.start()
            return r

        rxs, rys = [], []
        for g in range(2):
            for j in range(3):
                drx_ref[g, pl.ds(j * M_PER, M_PER), :] = cdot(
                    slot(4 * g + j) * SQ + qd)
            dry_ref[g, :, :] = cdot(slot(4 * g + 3) * SQ + qd)
            if g == 0:
                pl.semaphore_wait(bar, N_DEV - 1)
            rxs.append(copy(drx_ref.at[g], in_rx_ref.at[g],
                            s_rx.at[g], r_rx.at[g], dev1))
            rys.append(copy(dry_ref.at[g], in_ry_ref.at[g],
                            s_ry.at[g], r_ry.at[g], dev2))

        mbs, mcs = [None, None], [None, None]
        for g in range(2):
            for j in range(4):
                mb_ref[g, pl.ds(j * M_PER, M_PER), :] = cdot(
                    slot(4 * g + j) * SQ + q1)
            rys[g].wait_recv()
            mb_ref[g, pl.ds(3 * M_PER, M_PER), :] = (
                mb_ref[g, pl.ds(3 * M_PER, M_PER), :] + in_ry_ref[g])
            mbs[g] = copy(mb_ref.at[g], in_ma1_ref.at[g],
                          s_mb.at[g], r_ma1.at[g], dev1)
            for j in range(4):
                mc_ref[g, pl.ds(j * M_PER, M_PER), :] = cdot(
                    slot(4 * g + j) * SQ + q2)
        for g in range(2):
            rxs[g].wait_recv()
            mc_ref[g, pl.ds(0, 3 * M_PER), :] = (
                mc_ref[g, pl.ds(0, 3 * M_PER), :] + in_rx_ref[g])
            mcs[g] = copy(mc_ref.at[g], in_ma2_ref.at[g],
                          s_mc.at[g], r_ma2.at[g], dev2)

        for p in range(NG):
            work_ref[pl.ds(p * M_PER, M_PER), :] = cdot(slot(p) * SQ + q)

        rdma2 = {}
        for g in range(2):
            mbs[g].wait_recv()
            mcs[g].wait_recv()
            work_ref[pl.ds(g * H * M_PER, H * M_PER), :] = (
                work_ref[pl.ds(g * H * M_PER, H * M_PER), :]
                + in_ma1_ref[g] + in_ma2_ref[g])
            for u in range(1 + g * H, min(1 + (g + 1) * H, NG)):
                rdma2[u] = copy(
                    work_ref.at[pl.ds((u - 1) * M_PER, M_PER), :],
                    inbox2_ref.at[NG - 1 - u],
                    send2.at[u - 1], recv2.at[NG - 1 - u],
                    slot(u - 1) * SQ + q)

        result = work_ref[pl.ds((NG - 1) * M_PER, M_PER), :]
        for u in range(1, NG):
            rdma2[u].wait_recv()
            result = result + inbox2_ref[NG - 1 - u]
        out_ref[...] = jnp.maximum(result, 0.0)

        for r in rxs + rys + mbs + mcs + list(rdma2.values()):
            r.wait_send()

    return pl.pallas_call(
        body,
        out_shape=jax.ShapeDtypeStruct((M_PER, n), jnp.float32),
        in_specs=[
            pl.BlockSpec(memory_space=pltpu.VMEM),
            pl.BlockSpec(memory_space=pltpu.VMEM),
        ],
        out_specs=pl.BlockSpec(memory_space=pltpu.VMEM),
        scratch_shapes=[
            pltpu.VMEM((2, 3 * M_PER, n), jnp.float32),
            pltpu.VMEM((2, M_PER, n), jnp.float32),
            pltpu.VMEM((2, H * M_PER, n), jnp.float32),
            pltpu.VMEM((2, H * M_PER, n), jnp.float32),
            pltpu.VMEM((NG * M_PER, n), jnp.float32),
            pltpu.VMEM((2, 3 * M_PER, n), jnp.float32),
            pltpu.VMEM((2, M_PER, n), jnp.float32),
            pltpu.VMEM((2, H * M_PER, n), jnp.float32),
            pltpu.VMEM((2, H * M_PER, n), jnp.float32),
            pltpu.VMEM((NG - 1, M_PER, n), jnp.float32),
            pltpu.SemaphoreType.DMA((2,)),
            pltpu.SemaphoreType.DMA((2,)),
            pltpu.SemaphoreType.DMA((2,)),
            pltpu.SemaphoreType.DMA((2,)),
            pltpu.SemaphoreType.DMA((2,)),
            pltpu.SemaphoreType.DMA((2,)),
            pltpu.SemaphoreType.DMA((2,)),
            pltpu.SemaphoreType.DMA((2,)),
            pltpu.SemaphoreType.DMA((NG - 1,)),
            pltpu.SemaphoreType.DMA((NG - 1,)),
        ],
        compiler_params=pltpu.CompilerParams(collective_id=0),
    )(x, w_mat)
